# baseline (speedup 1.0000x reference)
import functools

import jax
import jax.numpy as jnp
from jax import lax
from jax.experimental import pallas as pl
from jax.experimental.pallas import tpu as pltpu
from jax.experimental.pallas import tpu_sc as plsc

N = 10000
N_PAD = 10240
E = 160000
NW = 32
BLK = 32
E_BLKS = E // BLK
E_PAD = 174080
TBLK = E_PAD // BLK
NBLK = TBLK // NW
ROWS_PER_SUB = N_PAD // 16
NBA = TBLK // 16

_MESH = plsc.VectorSubcoreMesh(core_axis_name="c", subcore_axis_name="s")


def _deg_body(row_hbm, col_hbm, ew_hbm, out_hbm, row_v, col_v, ew_v,
              ewe_v, msg_v, zero_v, acc):
    c = lax.axis_index("c")
    s = lax.axis_index("s")
    wid = c * 16 + s
    pltpu.sync_copy(row_hbm.at[pl.ds(wid * NBLK, NBLK)], row_v)
    pltpu.sync_copy(col_hbm.at[pl.ds(wid * NBLK, NBLK)], col_v)
    pltpu.sync_copy(ew_hbm.at[pl.ds(wid * NBLK, NBLK)], ew_v)

    def zb(i, carry):
        zero_v[i, pl.ds(0, 16)] = jnp.zeros((16,), jnp.float32)
        return carry
    lax.fori_loop(0, ROWS_PER_SUB, zb, 0)
    pltpu.sync_copy(zero_v, acc.at[pl.ds(s * ROWS_PER_SUB, ROWS_PER_SUB)])
    plsc.subcore_barrier()

    def blk(b, carry):
        orig = (wid * NBLK + b) < E_BLKS
        for g in range(BLK // 16):
            rv = row_v[b, pl.ds(g * 16, 16)]
            cv = col_v[b, pl.ds(g * 16, 16)]
            wv = ew_v[b, pl.ds(g * 16, 16)]
            ewe_v[pl.ds(g * 16, 16)] = jnp.where((rv == cv) & orig, 0.0, wv)

        def edge(e, carry2):
            sp = plsc.load_gather(ewe_v, [jnp.full((16,), e, jnp.int32)])
            msg_v[e, pl.ds(0, 16)] = sp
            return carry2
        lax.fori_loop(0, BLK, edge, 0)
        pltpu.sync_copy(msg_v, acc.at[row_v.at[b]], add=True)
        return carry
    lax.fori_loop(0, NBLK, blk, 0)
    plsc.subcore_barrier()
    pltpu.sync_copy(acc.at[pl.ds(s * ROWS_PER_SUB, ROWS_PER_SUB)],
                    out_hbm.at[c, pl.ds(s * ROWS_PER_SUB, ROWS_PER_SUB)])


_prep_deg = pl.kernel(
    _deg_body,
    out_type=jax.ShapeDtypeStruct((2, N_PAD, 16), jnp.float32),
    mesh=_MESH,
    compiler_params=pltpu.CompilerParams(needs_layout_passes=False, use_tc_tiling_on_sc=False),
    scratch_types=[
        pltpu.VMEM((NBLK, BLK), jnp.int32),
        pltpu.VMEM((NBLK, BLK), jnp.int32),
        pltpu.VMEM((NBLK, BLK), jnp.float32),
        pltpu.VMEM((BLK,), jnp.float32),
        pltpu.VMEM((BLK, 16), jnp.float32),
        pltpu.VMEM((ROWS_PER_SUB, 16), jnp.float32),
        pltpu.VMEM_SHARED((N_PAD, 16), jnp.float32),
    ],
)


def _norm_body(row_hbm, col_hbm, ew_hbm, dis_hbm, out_hbm,
               row_v, col_v, ew_v, dis_v, norm_v):
    c = lax.axis_index("c")
    s = lax.axis_index("s")
    wid = c * 16 + s
    pltpu.sync_copy(row_hbm.at[pl.ds(wid * NBLK, NBLK)], row_v)
    pltpu.sync_copy(col_hbm.at[pl.ds(wid * NBLK, NBLK)], col_v)
    pltpu.sync_copy(ew_hbm.at[pl.ds(wid * NBLK, NBLK)], ew_v)
    pltpu.sync_copy(dis_hbm, dis_v)

    z16 = jnp.zeros((16,), jnp.int32)

    def blk(b, carry):
        orig = (wid * NBLK + b) < E_BLKS
        for g in range(BLK // 16):
            rv = row_v[b, pl.ds(g * 16, 16)]
            cv = col_v[b, pl.ds(g * 16, 16)]
            wv = ew_v[b, pl.ds(g * 16, 16)]
            ewe = jnp.where((rv == cv) & orig, 0.0, wv)
            dr = plsc.load_gather(dis_v, [rv, z16])
            dc = plsc.load_gather(dis_v, [cv, z16])
            norm_v[b, pl.ds(g * 16, 16)] = dr * ewe * dc
        return carry
    lax.fori_loop(0, NBLK, blk, 0)
    pltpu.sync_copy(norm_v, out_hbm.at[pl.ds(wid * NBLK, NBLK)])


_prep_norm = pl.kernel(
    _norm_body,
    out_type=jax.ShapeDtypeStruct((TBLK, BLK), jnp.float32),
    mesh=_MESH,
    compiler_params=pltpu.CompilerParams(needs_layout_passes=False, use_tc_tiling_on_sc=False),
    scratch_types=[
        pltpu.VMEM((NBLK, BLK), jnp.int32),
        pltpu.VMEM((NBLK, BLK), jnp.int32),
        pltpu.VMEM((NBLK, BLK), jnp.float32),
        pltpu.VMEM((N_PAD, 1), jnp.float32),
        pltpu.VMEM((NBLK, BLK), jnp.float32),
    ],
)


def _agg_body(fc_chunks, h_hbm, row_hbm, col_hbm, norm_hbm, out_hbm,
              row_v, col_v, norm_v, msg_v, acc, *sems):
    c = lax.axis_index("c")
    s = lax.axis_index("s")
    sg, ss = sems[:4], sems[4:]
    base = s * NBA
    pltpu.sync_copy(row_hbm.at[pl.ds(base, NBA)], row_v)
    pltpu.sync_copy(col_hbm.at[pl.ds(base, NBA)], col_v)
    pltpu.sync_copy(norm_hbm.at[pl.ds(base, NBA)], norm_v)

    def scale_block(buf, b):
        def edge(e2, carry2):
            for u in range(2):
                e = e2 * 2 + u
                sp = plsc.load_gather(
                    norm_v, [jnp.full((16,), b, jnp.int32),
                             jnp.full((16,), e, jnp.int32)])
                for j in range(128 // 16):
                    seg = msg_v[buf, e, pl.ds(j * 16, 16)]
                    msg_v[buf, e, pl.ds(j * 16, 16)] = seg * sp
            return carry2
        lax.fori_loop(0, BLK // 2, edge, 0)

    def start_gather(fc, b, buf, sem):
        pltpu.async_copy(h_hbm.at[fc].at[row_v.at[b]], msg_v.at[buf], sem)

    def wait_gather(fc, b, buf, sem):
        pltpu.make_async_copy(h_hbm.at[fc].at[row_v.at[b]], msg_v.at[buf],
                              sem).wait()

    def start_scatter(b, buf, sem):
        pltpu.async_copy(msg_v.at[buf], acc.at[col_v.at[b]], sem, add=True)

    def wait_scatter(b, buf, sem):
        pltpu.make_async_copy(msg_v.at[buf], acc.at[col_v.at[b]], sem).wait()

    def zero_phase():
        def zb(i, carry):
            for g in range(8):
                msg_v[0, i, pl.ds(g * 16, 16)] = jnp.zeros((16,), jnp.float32)
            return carry
        lax.fori_loop(0, BLK, zb, 0)
        for z in range(ROWS_PER_SUB // BLK):
            pltpu.async_copy(msg_v.at[0],
                             acc.at[pl.ds(s * ROWS_PER_SUB + z * BLK, BLK)],
                             sg[0])
        for z in range(ROWS_PER_SUB // BLK):
            pltpu.make_async_copy(
                msg_v.at[0],
                acc.at[pl.ds(s * ROWS_PER_SUB + z * BLK, BLK)],
                sg[0]).wait()

    def edge_phase(fc):
        for k in range(3):
            start_gather(fc, k, k, sg[k])

        def blk(p, carry):
            for u in range(4):
                k = p * 4 + u
                v = (u + 3) % 4
                wait_gather(fc, k, u, sg[u])
                scale_block(u, k)
                start_scatter(k, u, ss[u])

                @pl.when(k > 0)
                def _():
                    wait_scatter(k - 1, v, ss[v])

                @pl.when(k + 3 < NBA)
                def _():
                    start_gather(fc, k + 3, v, sg[v])
            return carry
        lax.fori_loop(0, NBA // 4, blk, 0)
        wait_scatter(NBA - 1, 3, ss[3])

    def drain_phase(fc):
        pltpu.sync_copy(acc.at[pl.ds(s * ROWS_PER_SUB, ROWS_PER_SUB)],
                        out_hbm.at[fc, pl.ds(s * ROWS_PER_SUB,
                                             ROWS_PER_SUB)])

    c0_fcs = list(range(fc_chunks - 1))
    c1_fcs = [fc_chunks - 1]
    for i in range(len(c0_fcs)):
        have1 = i < len(c1_fcs)

        @pl.when(c == 0)
        def _():
            zero_phase()
        if have1:
            @pl.when(c == 1)
            def _():
                zero_phase()
        plsc.subcore_barrier()

        @pl.when(c == 0)
        def _():
            edge_phase(c0_fcs[i])
        if have1:
            @pl.when(c == 1)
            def _():
                edge_phase(c1_fcs[i])
        plsc.subcore_barrier()

        @pl.when(c == 0)
        def _():
            drain_phase(c0_fcs[i])

    @pl.when(c == 1)
    def _():
        drain_phase(c1_fcs[0])


def _make_agg(fc_chunks):
    return pl.kernel(
        functools.partial(_agg_body, fc_chunks),
        out_type=jax.ShapeDtypeStruct((fc_chunks, N_PAD, 128),
                                      jnp.float32),
        mesh=_MESH,
        compiler_params=pltpu.CompilerParams(needs_layout_passes=False, use_tc_tiling_on_sc=False),
        scratch_types=[
            pltpu.VMEM((NBA, BLK), jnp.int32),
            pltpu.VMEM((NBA, BLK), jnp.int32),
            pltpu.VMEM((NBA, BLK), jnp.float32),
            pltpu.VMEM((4, BLK, 128), jnp.float32),
            pltpu.VMEM_SHARED((N_PAD, 128), jnp.float32),
        ] + [pltpu.SemaphoreType.DMA] * 8,
    )


_agg4 = _make_agg(4)
_agg2 = _make_agg(2)


def _dis_body(degw_ref, dis_ref):
    deg = degw_ref[0][:, 0:1] + degw_ref[1][:, 0:1]
    dis_ref[...] = jnp.where(deg > 0,
                             lax.rsqrt(jnp.where(deg > 0, deg, 1.0)), 0.0)


def _dis_kernel(degw):
    return pl.pallas_call(
        _dis_body,
        out_shape=jax.ShapeDtypeStruct((N_PAD, 1), jnp.float32),
    )(degw)


def _mm_body(x_ref, w_ref, o_ref):
    o_ref[0] = jnp.dot(x_ref[...], w_ref[...],
                       preferred_element_type=jnp.float32)


def _matmul_fc(x, w):
    k, dout = w.shape
    fc_chunks = dout // 128
    tm = 512
    grid = (N_PAD // tm, fc_chunks)
    return pl.pallas_call(
        _mm_body,
        grid=grid,
        in_specs=[
            pl.BlockSpec((tm, k), lambda m, f: (m, 0)),
            pl.BlockSpec((k, 128), lambda m, f: (0, f)),
        ],
        out_specs=pl.BlockSpec((1, tm, 128), lambda m, f: (f, m, 0)),
        out_shape=jax.ShapeDtypeStruct((fc_chunks, N_PAD, 128), jnp.float32),
    )(x, w)


def _epi_body(relu, agg_ref, b_ref, o_ref):
    val = agg_ref[0] + b_ref[...]
    if relu:
        val = jnp.maximum(val, 0.0)
    o_ref[...] = val


def _epilogue(aggp, bias2d, relu):
    fc_chunks = aggp.shape[0]
    dout = fc_chunks * 128
    tm = 512
    grid = (N_PAD // tm, fc_chunks)
    return pl.pallas_call(
        functools.partial(_epi_body, relu),
        grid=grid,
        in_specs=[
            pl.BlockSpec((1, tm, 128), lambda m, f: (f, m, 0)),
            pl.BlockSpec((1, 128), lambda m, f: (0, f)),
        ],
        out_specs=pl.BlockSpec((tm, 128), lambda m, f: (m, f)),
        out_shape=jax.ShapeDtypeStruct((N_PAD, dout), jnp.float32),
    )(aggp, bias2d)


def kernel(x, edge_index, edge_weights, W1, b1, W2, b2, W3, b3):
    loop = jnp.arange(N_PAD, dtype=jnp.int32)
    tail = jnp.zeros((E_PAD - E - N_PAD,), jnp.int32)
    row = jnp.concatenate([edge_index[0], loop, tail])
    col = jnp.concatenate([edge_index[1], loop, tail])
    ew = jnp.concatenate([edge_weights, jnp.ones((N_PAD,), jnp.float32),
                          tail.astype(jnp.float32)])
    row = row.reshape(TBLK, BLK)
    col = col.reshape(TBLK, BLK)
    ew = ew.reshape(TBLK, BLK)
    xp = jnp.pad(x, ((0, N_PAD - N), (0, 0)))

    degw = _prep_deg(row, col, ew)
    dis = _dis_kernel(degw)
    norm = _prep_norm(row, col, ew, dis)

    h = xp
    for (w, b, agg_k, relu) in ((W1, b1, _agg4, True),
                                (W2, b2, _agg4, True),
                                (W3, b3, _agg2, False)):
        hw = _matmul_fc(h, w)
        aggp = agg_k(hw, row, col, norm)
        h = _epilogue(aggp, b.reshape(1, -1), relu)

    return h[:N]


# speedup vs baseline: 1.2854x; 1.2854x over previous
import functools

import jax
import jax.numpy as jnp
from jax import lax
from jax.experimental import pallas as pl
from jax.experimental.pallas import tpu as pltpu
from jax.experimental.pallas import tpu_sc as plsc

N = 10000
N_PAD = 10240
E = 160000
NW = 32
BLK = 32
E_BLKS = E // BLK
E_PAD = 174080
TBLK = E_PAD // BLK
NBLK = TBLK // NW
ROWS_PER_SUB = N_PAD // 16
NBA = TBLK // 16

_MESH = plsc.VectorSubcoreMesh(core_axis_name="c", subcore_axis_name="s")


def _deg_body(row_hbm, col_hbm, ew_hbm, out_hbm, row_v, col_v, ew_v,
              ewe_v, msg_v, zero_v, acc):
    c = lax.axis_index("c")
    s = lax.axis_index("s")
    wid = c * 16 + s
    pltpu.sync_copy(row_hbm.at[pl.ds(wid * NBLK, NBLK)], row_v)
    pltpu.sync_copy(col_hbm.at[pl.ds(wid * NBLK, NBLK)], col_v)
    pltpu.sync_copy(ew_hbm.at[pl.ds(wid * NBLK, NBLK)], ew_v)

    def zb(i, carry):
        zero_v[i, pl.ds(0, 16)] = jnp.zeros((16,), jnp.float32)
        return carry
    lax.fori_loop(0, ROWS_PER_SUB, zb, 0)
    pltpu.sync_copy(zero_v, acc.at[pl.ds(s * ROWS_PER_SUB, ROWS_PER_SUB)])
    plsc.subcore_barrier()

    def blk(b, carry):
        orig = (wid * NBLK + b) < E_BLKS
        for g in range(BLK // 16):
            rv = row_v[b, pl.ds(g * 16, 16)]
            cv = col_v[b, pl.ds(g * 16, 16)]
            wv = ew_v[b, pl.ds(g * 16, 16)]
            ewe_v[pl.ds(g * 16, 16)] = jnp.where((rv == cv) & orig, 0.0, wv)

        def edge(e, carry2):
            sp = plsc.load_gather(ewe_v, [jnp.full((16,), e, jnp.int32)])
            msg_v[e, pl.ds(0, 16)] = sp
            return carry2
        lax.fori_loop(0, BLK, edge, 0)
        pltpu.sync_copy(msg_v, acc.at[row_v.at[b]], add=True)
        return carry
    lax.fori_loop(0, NBLK, blk, 0)
    plsc.subcore_barrier()
    pltpu.sync_copy(acc.at[pl.ds(s * ROWS_PER_SUB, ROWS_PER_SUB)],
                    out_hbm.at[c, pl.ds(s * ROWS_PER_SUB, ROWS_PER_SUB)])


_prep_deg = pl.kernel(
    _deg_body,
    out_type=jax.ShapeDtypeStruct((2, N_PAD, 16), jnp.float32),
    mesh=_MESH,
    compiler_params=pltpu.CompilerParams(needs_layout_passes=False, use_tc_tiling_on_sc=False),
    scratch_types=[
        pltpu.VMEM((NBLK, BLK), jnp.int32),
        pltpu.VMEM((NBLK, BLK), jnp.int32),
        pltpu.VMEM((NBLK, BLK), jnp.float32),
        pltpu.VMEM((BLK,), jnp.float32),
        pltpu.VMEM((BLK, 16), jnp.float32),
        pltpu.VMEM((ROWS_PER_SUB, 16), jnp.float32),
        pltpu.VMEM_SHARED((N_PAD, 16), jnp.float32),
    ],
)


def _norm_body(row_hbm, col_hbm, ew_hbm, dis_hbm, out_hbm,
               row_v, col_v, ew_v, dis_v, norm_v):
    c = lax.axis_index("c")
    s = lax.axis_index("s")
    wid = c * 16 + s
    pltpu.sync_copy(row_hbm.at[pl.ds(wid * NBLK, NBLK)], row_v)
    pltpu.sync_copy(col_hbm.at[pl.ds(wid * NBLK, NBLK)], col_v)
    pltpu.sync_copy(ew_hbm.at[pl.ds(wid * NBLK, NBLK)], ew_v)
    pltpu.sync_copy(dis_hbm, dis_v)

    z16 = jnp.zeros((16,), jnp.int32)

    def blk(b, carry):
        orig = (wid * NBLK + b) < E_BLKS
        for g in range(BLK // 16):
            rv = row_v[b, pl.ds(g * 16, 16)]
            cv = col_v[b, pl.ds(g * 16, 16)]
            wv = ew_v[b, pl.ds(g * 16, 16)]
            ewe = jnp.where((rv == cv) & orig, 0.0, wv)
            dr = plsc.load_gather(dis_v, [rv, z16])
            dc = plsc.load_gather(dis_v, [cv, z16])
            norm_v[b, pl.ds(g * 16, 16)] = dr * ewe * dc
        return carry
    lax.fori_loop(0, NBLK, blk, 0)
    pltpu.sync_copy(norm_v, out_hbm.at[pl.ds(wid * NBLK, NBLK)])


_prep_norm = pl.kernel(
    _norm_body,
    out_type=jax.ShapeDtypeStruct((TBLK, BLK), jnp.float32),
    mesh=_MESH,
    compiler_params=pltpu.CompilerParams(needs_layout_passes=False, use_tc_tiling_on_sc=False),
    scratch_types=[
        pltpu.VMEM((NBLK, BLK), jnp.int32),
        pltpu.VMEM((NBLK, BLK), jnp.int32),
        pltpu.VMEM((NBLK, BLK), jnp.float32),
        pltpu.VMEM((N_PAD, 1), jnp.float32),
        pltpu.VMEM((NBLK, BLK), jnp.float32),
    ],
)


def _agg_body(fc_chunks, h_hbm, row_hbm, col_hbm, norm_hbm, out_hbm,
              row_v, col_v, norm_v, msg_v, acc, *sems):
    c = lax.axis_index("c")
    s = lax.axis_index("s")
    sg, ss = sems[:4], sems[4:]
    base = s * NBA
    pltpu.sync_copy(row_hbm.at[pl.ds(base, NBA)], row_v)
    pltpu.sync_copy(col_hbm.at[pl.ds(base, NBA)], col_v)
    pltpu.sync_copy(norm_hbm.at[pl.ds(base, NBA)], norm_v)

    def scale_block(buf, b):
        def edge(e2, carry2):
            for u in range(2):
                e = e2 * 2 + u
                sp = plsc.load_gather(
                    norm_v, [jnp.full((16,), b, jnp.int32),
                             jnp.full((16,), e, jnp.int32)])
                for j in range(128 // 16):
                    seg = msg_v[buf, e, pl.ds(j * 16, 16)]
                    msg_v[buf, e, pl.ds(j * 16, 16)] = seg * sp
            return carry2
        lax.fori_loop(0, BLK // 2, edge, 0)

    def start_gather(fc, b, buf, sem):
        pltpu.async_copy(h_hbm.at[fc].at[row_v.at[b]], msg_v.at[buf], sem)

    def wait_gather(fc, b, buf, sem):
        pltpu.make_async_copy(h_hbm.at[fc].at[row_v.at[b]], msg_v.at[buf],
                              sem).wait()

    def start_scatter(b, buf, sem):
        pltpu.async_copy(msg_v.at[buf], acc.at[col_v.at[b]], sem, add=True)

    def wait_scatter(b, buf, sem):
        pltpu.make_async_copy(msg_v.at[buf], acc.at[col_v.at[b]], sem).wait()

    def zero_phase():
        def zb(i, carry):
            for g in range(8):
                msg_v[0, i, pl.ds(g * 16, 16)] = jnp.zeros((16,), jnp.float32)
            return carry
        lax.fori_loop(0, BLK, zb, 0)
        for z in range(ROWS_PER_SUB // BLK):
            pltpu.async_copy(msg_v.at[0],
                             acc.at[pl.ds(s * ROWS_PER_SUB + z * BLK, BLK)],
                             sg[0])
        for z in range(ROWS_PER_SUB // BLK):
            pltpu.make_async_copy(
                msg_v.at[0],
                acc.at[pl.ds(s * ROWS_PER_SUB + z * BLK, BLK)],
                sg[0]).wait()

    def edge_phase(fc):
        for k in range(3):
            start_gather(fc, k, k, sg[k])

        def blk(p, carry):
            for u in range(4):
                k = p * 4 + u
                v = (u + 3) % 4
                wait_gather(fc, k, u, sg[u])
                scale_block(u, k)
                start_scatter(k, u, ss[u])

                @pl.when(k > 0)
                def _():
                    wait_scatter(k - 1, v, ss[v])

                @pl.when(k + 3 < NBA)
                def _():
                    start_gather(fc, k + 3, v, sg[v])
            return carry
        lax.fori_loop(0, NBA // 4, blk, 0)
        wait_scatter(NBA - 1, 3, ss[3])

    def drain_phase(fc):
        pltpu.sync_copy(acc.at[pl.ds(s * ROWS_PER_SUB, ROWS_PER_SUB)],
                        out_hbm.at[fc, pl.ds(s * ROWS_PER_SUB,
                                             ROWS_PER_SUB)])

    half = fc_chunks // 2
    c0_fcs = list(range(half))
    c1_fcs = list(range(half, fc_chunks))
    for i in range(len(c0_fcs)):
        @pl.when(c == 0)
        def _():
            zero_phase()

        @pl.when(c == 1)
        def _():
            zero_phase()
        plsc.subcore_barrier()

        @pl.when(c == 0)
        def _():
            edge_phase(c0_fcs[i])

        @pl.when(c == 1)
        def _():
            edge_phase(c1_fcs[i])
        plsc.subcore_barrier()

        @pl.when(c == 0)
        def _():
            drain_phase(c0_fcs[i])
        if i < len(c1_fcs) - 1:
            @pl.when(c == 1)
            def _():
                drain_phase(c1_fcs[i])

    @pl.when(c == 1)
    def _():
        drain_phase(c1_fcs[-1])


def _make_agg(fc_chunks):
    return pl.kernel(
        functools.partial(_agg_body, fc_chunks),
        out_type=jax.ShapeDtypeStruct((fc_chunks, N_PAD, 128),
                                      jnp.float32),
        mesh=_MESH,
        compiler_params=pltpu.CompilerParams(needs_layout_passes=False, use_tc_tiling_on_sc=False),
        scratch_types=[
            pltpu.VMEM((NBA, BLK), jnp.int32),
            pltpu.VMEM((NBA, BLK), jnp.int32),
            pltpu.VMEM((NBA, BLK), jnp.float32),
            pltpu.VMEM((4, BLK, 128), jnp.float32),
            pltpu.VMEM_SHARED((N_PAD, 128), jnp.float32),
        ] + [pltpu.SemaphoreType.DMA] * 8,
    )


_agg4 = _make_agg(4)
_agg2 = _make_agg(2)


def _dis_body(degw_ref, dis_ref):
    deg = degw_ref[0][:, 0:1] + degw_ref[1][:, 0:1]
    dis_ref[...] = jnp.where(deg > 0,
                             lax.rsqrt(jnp.where(deg > 0, deg, 1.0)), 0.0)


def _dis_kernel(degw):
    return pl.pallas_call(
        _dis_body,
        out_shape=jax.ShapeDtypeStruct((N_PAD, 1), jnp.float32),
    )(degw)


def _mm_body(x_ref, w_ref, o_ref):
    o_ref[0] = jnp.dot(x_ref[...], w_ref[...],
                       preferred_element_type=jnp.float32)


def _matmul_fc(x, w):
    k, dout = w.shape
    fc_chunks = dout // 128
    tm = 512
    grid = (N_PAD // tm, fc_chunks)
    return pl.pallas_call(
        _mm_body,
        grid=grid,
        in_specs=[
            pl.BlockSpec((tm, k), lambda m, f: (m, 0)),
            pl.BlockSpec((k, 128), lambda m, f: (0, f)),
        ],
        out_specs=pl.BlockSpec((1, tm, 128), lambda m, f: (f, m, 0)),
        out_shape=jax.ShapeDtypeStruct((fc_chunks, N_PAD, 128), jnp.float32),
    )(x, w)


def _epi_body(relu, agg_ref, b_ref, o_ref):
    val = agg_ref[0] + b_ref[...]
    if relu:
        val = jnp.maximum(val, 0.0)
    o_ref[...] = val


def _epilogue(aggp, bias2d, relu):
    fc_chunks = aggp.shape[0]
    dout = fc_chunks * 128
    tm = 512
    grid = (N_PAD // tm, fc_chunks)
    return pl.pallas_call(
        functools.partial(_epi_body, relu),
        grid=grid,
        in_specs=[
            pl.BlockSpec((1, tm, 128), lambda m, f: (f, m, 0)),
            pl.BlockSpec((1, 128), lambda m, f: (0, f)),
        ],
        out_specs=pl.BlockSpec((tm, 128), lambda m, f: (m, f)),
        out_shape=jax.ShapeDtypeStruct((N_PAD, dout), jnp.float32),
    )(aggp, bias2d)


def kernel(x, edge_index, edge_weights, W1, b1, W2, b2, W3, b3):
    loop = jnp.arange(N_PAD, dtype=jnp.int32)
    tail = jnp.zeros((E_PAD - E - N_PAD,), jnp.int32)
    row = jnp.concatenate([edge_index[0], loop, tail])
    col = jnp.concatenate([edge_index[1], loop, tail])
    ew = jnp.concatenate([edge_weights, jnp.ones((N_PAD,), jnp.float32),
                          tail.astype(jnp.float32)])
    row = row.reshape(TBLK, BLK)
    col = col.reshape(TBLK, BLK)
    ew = ew.reshape(TBLK, BLK)
    xp = jnp.pad(x, ((0, N_PAD - N), (0, 0)))

    degw = _prep_deg(row, col, ew)
    dis = _dis_kernel(degw)
    norm = _prep_norm(row, col, ew, dis)

    h = xp
    for (w, b, agg_k, relu) in ((W1, b1, _agg4, True),
                                (W2, b2, _agg4, True),
                                (W3, b3, _agg2, False)):
        hw = _matmul_fc(h, w)
        aggp = agg_k(hw, row, col, norm)
        h = _epilogue(aggp, b.reshape(1, -1), relu)

    return h[:N]


# speedup vs baseline: 1.3038x; 1.0143x over previous
import functools

import jax
import jax.numpy as jnp
from jax import lax
from jax.experimental import pallas as pl
from jax.experimental.pallas import tpu as pltpu
from jax.experimental.pallas import tpu_sc as plsc

N = 10000
N_PAD = 10240
E = 160000
NW = 32
BLK = 32
E_BLKS = E // BLK
E_PAD = 174080
TBLK = E_PAD // BLK
NBLK = TBLK // NW
ROWS_PER_SUB = N_PAD // 16
NBA = TBLK // 16

_MESH = plsc.VectorSubcoreMesh(core_axis_name="c", subcore_axis_name="s")


def _deg_body(row_hbm, col_hbm, ew_hbm, out_hbm, row_v, col_v, ew_v,
              ewe_v, msg_v, zero_v, acc):
    c = lax.axis_index("c")
    s = lax.axis_index("s")
    wid = c * 16 + s
    pltpu.sync_copy(row_hbm.at[pl.ds(wid * NBLK, NBLK)], row_v)
    pltpu.sync_copy(col_hbm.at[pl.ds(wid * NBLK, NBLK)], col_v)
    pltpu.sync_copy(ew_hbm.at[pl.ds(wid * NBLK, NBLK)], ew_v)

    def zb(i, carry):
        zero_v[i, pl.ds(0, 16)] = jnp.zeros((16,), jnp.float32)
        return carry
    lax.fori_loop(0, ROWS_PER_SUB, zb, 0)
    pltpu.sync_copy(zero_v, acc.at[pl.ds(s * ROWS_PER_SUB, ROWS_PER_SUB)])
    plsc.subcore_barrier()

    def blk(b, carry):
        orig = (wid * NBLK + b) < E_BLKS
        for g in range(BLK // 16):
            rv = row_v[b, pl.ds(g * 16, 16)]
            cv = col_v[b, pl.ds(g * 16, 16)]
            wv = ew_v[b, pl.ds(g * 16, 16)]
            ewe_v[pl.ds(g * 16, 16)] = jnp.where((rv == cv) & orig, 0.0, wv)

        def edge(e, carry2):
            sp = plsc.load_gather(ewe_v, [jnp.full((16,), e, jnp.int32)])
            msg_v[e, pl.ds(0, 16)] = sp
            return carry2
        lax.fori_loop(0, BLK, edge, 0)
        pltpu.sync_copy(msg_v, acc.at[row_v.at[b]], add=True)
        return carry
    lax.fori_loop(0, NBLK, blk, 0)
    plsc.subcore_barrier()
    pltpu.sync_copy(acc.at[pl.ds(s * ROWS_PER_SUB, ROWS_PER_SUB)],
                    out_hbm.at[c, pl.ds(s * ROWS_PER_SUB, ROWS_PER_SUB)])


_prep_deg = pl.kernel(
    _deg_body,
    out_type=jax.ShapeDtypeStruct((2, N_PAD, 16), jnp.float32),
    mesh=_MESH,
    compiler_params=pltpu.CompilerParams(needs_layout_passes=False, use_tc_tiling_on_sc=False),
    scratch_types=[
        pltpu.VMEM((NBLK, BLK), jnp.int32),
        pltpu.VMEM((NBLK, BLK), jnp.int32),
        pltpu.VMEM((NBLK, BLK), jnp.float32),
        pltpu.VMEM((BLK,), jnp.float32),
        pltpu.VMEM((BLK, 16), jnp.float32),
        pltpu.VMEM((ROWS_PER_SUB, 16), jnp.float32),
        pltpu.VMEM_SHARED((N_PAD, 16), jnp.float32),
    ],
)


def _norm_body(row_hbm, col_hbm, ew_hbm, dis_hbm, out_hbm,
               row_v, col_v, ew_v, dis_v, norm_v):
    c = lax.axis_index("c")
    s = lax.axis_index("s")
    wid = c * 16 + s
    pltpu.sync_copy(row_hbm.at[pl.ds(wid * NBLK, NBLK)], row_v)
    pltpu.sync_copy(col_hbm.at[pl.ds(wid * NBLK, NBLK)], col_v)
    pltpu.sync_copy(ew_hbm.at[pl.ds(wid * NBLK, NBLK)], ew_v)
    pltpu.sync_copy(dis_hbm, dis_v)

    z16 = jnp.zeros((16,), jnp.int32)

    def blk(b, carry):
        orig = (wid * NBLK + b) < E_BLKS
        for g in range(BLK // 16):
            rv = row_v[b, pl.ds(g * 16, 16)]
            cv = col_v[b, pl.ds(g * 16, 16)]
            wv = ew_v[b, pl.ds(g * 16, 16)]
            ewe = jnp.where((rv == cv) & orig, 0.0, wv)
            dr = plsc.load_gather(dis_v, [rv, z16])
            dc = plsc.load_gather(dis_v, [cv, z16])
            norm_v[b, pl.ds(g * 16, 16)] = dr * ewe * dc
        return carry
    lax.fori_loop(0, NBLK, blk, 0)
    pltpu.sync_copy(norm_v, out_hbm.at[pl.ds(wid * NBLK, NBLK)])


_prep_norm = pl.kernel(
    _norm_body,
    out_type=jax.ShapeDtypeStruct((TBLK, BLK), jnp.float32),
    mesh=_MESH,
    compiler_params=pltpu.CompilerParams(needs_layout_passes=False, use_tc_tiling_on_sc=False),
    scratch_types=[
        pltpu.VMEM((NBLK, BLK), jnp.int32),
        pltpu.VMEM((NBLK, BLK), jnp.int32),
        pltpu.VMEM((NBLK, BLK), jnp.float32),
        pltpu.VMEM((N_PAD, 1), jnp.float32),
        pltpu.VMEM((NBLK, BLK), jnp.float32),
    ],
)


def _agg_body(fc_chunks, h_hbm, row_hbm, col_hbm, norm_hbm, out_hbm,
              row_v, col_v, norm_v, msg_v, acc, *sems):
    c = lax.axis_index("c")
    s = lax.axis_index("s")
    sg, ss = sems[:4], sems[4:]
    base = s * NBA
    pltpu.sync_copy(row_hbm.at[pl.ds(base, NBA)], row_v)
    pltpu.sync_copy(col_hbm.at[pl.ds(base, NBA)], col_v)
    pltpu.sync_copy(norm_hbm.at[pl.ds(base, NBA)], norm_v)

    def scale_block(buf, b):
        def edge(e4, carry2):
            sps = [plsc.load_gather(
                norm_v, [jnp.full((16,), b, jnp.int32),
                         jnp.full((16,), e4 * 4 + u, jnp.int32)])
                for u in range(4)]
            for j in range(128 // 16):
                for u in range(4):
                    e = e4 * 4 + u
                    seg = msg_v[buf, e, pl.ds(j * 16, 16)]
                    msg_v[buf, e, pl.ds(j * 16, 16)] = seg * sps[u]
            return carry2
        lax.fori_loop(0, BLK // 4, edge, 0)

    def start_gather(fc, b, buf, sem):
        pltpu.async_copy(h_hbm.at[fc].at[row_v.at[b]], msg_v.at[buf], sem)

    def wait_gather(fc, b, buf, sem):
        pltpu.make_async_copy(h_hbm.at[fc].at[row_v.at[b]], msg_v.at[buf],
                              sem).wait()

    def start_scatter(b, buf, sem):
        pltpu.async_copy(msg_v.at[buf], acc.at[col_v.at[b]], sem, add=True)

    def wait_scatter(b, buf, sem):
        pltpu.make_async_copy(msg_v.at[buf], acc.at[col_v.at[b]], sem).wait()

    def zero_phase():
        def zb(i, carry):
            for g in range(8):
                msg_v[0, i, pl.ds(g * 16, 16)] = jnp.zeros((16,), jnp.float32)
            return carry
        lax.fori_loop(0, BLK, zb, 0)
        for z in range(ROWS_PER_SUB // BLK):
            pltpu.async_copy(msg_v.at[0],
                             acc.at[pl.ds(s * ROWS_PER_SUB + z * BLK, BLK)],
                             sg[0])
        for z in range(ROWS_PER_SUB // BLK):
            pltpu.make_async_copy(
                msg_v.at[0],
                acc.at[pl.ds(s * ROWS_PER_SUB + z * BLK, BLK)],
                sg[0]).wait()

    def edge_phase(fc):
        for k in range(3):
            start_gather(fc, k, k, sg[k])

        def blk(p, carry):
            for u in range(4):
                k = p * 4 + u
                v = (u + 3) % 4
                wait_gather(fc, k, u, sg[u])
                scale_block(u, k)
                start_scatter(k, u, ss[u])

                @pl.when(k > 0)
                def _():
                    wait_scatter(k - 1, v, ss[v])

                @pl.when(k + 3 < NBA)
                def _():
                    start_gather(fc, k + 3, v, sg[v])
            return carry
        lax.fori_loop(0, NBA // 4, blk, 0)
        wait_scatter(NBA - 1, 3, ss[3])

    def drain_phase(fc):
        pltpu.sync_copy(acc.at[pl.ds(s * ROWS_PER_SUB, ROWS_PER_SUB)],
                        out_hbm.at[fc, pl.ds(s * ROWS_PER_SUB,
                                             ROWS_PER_SUB)])

    half = fc_chunks // 2
    c0_fcs = list(range(half))
    c1_fcs = list(range(half, fc_chunks))
    for i in range(len(c0_fcs)):
        @pl.when(c == 0)
        def _():
            zero_phase()

        @pl.when(c == 1)
        def _():
            zero_phase()
        plsc.subcore_barrier()

        @pl.when(c == 0)
        def _():
            edge_phase(c0_fcs[i])

        @pl.when(c == 1)
        def _():
            edge_phase(c1_fcs[i])
        plsc.subcore_barrier()

        @pl.when(c == 0)
        def _():
            drain_phase(c0_fcs[i])
        if i < len(c1_fcs) - 1:
            @pl.when(c == 1)
            def _():
                drain_phase(c1_fcs[i])

    @pl.when(c == 1)
    def _():
        drain_phase(c1_fcs[-1])


def _make_agg(fc_chunks):
    return pl.kernel(
        functools.partial(_agg_body, fc_chunks),
        out_type=jax.ShapeDtypeStruct((fc_chunks, N_PAD, 128),
                                      jnp.float32),
        mesh=_MESH,
        compiler_params=pltpu.CompilerParams(needs_layout_passes=False, use_tc_tiling_on_sc=False),
        scratch_types=[
            pltpu.VMEM((NBA, BLK), jnp.int32),
            pltpu.VMEM((NBA, BLK), jnp.int32),
            pltpu.VMEM((NBA, BLK), jnp.float32),
            pltpu.VMEM((4, BLK, 128), jnp.float32),
            pltpu.VMEM_SHARED((N_PAD, 128), jnp.float32),
        ] + [pltpu.SemaphoreType.DMA] * 8,
    )


_agg4 = _make_agg(4)
_agg2 = _make_agg(2)


def _dis_body(degw_ref, dis_ref):
    deg = degw_ref[0][:, 0:1] + degw_ref[1][:, 0:1]
    dis_ref[...] = jnp.where(deg > 0,
                             lax.rsqrt(jnp.where(deg > 0, deg, 1.0)), 0.0)


def _dis_kernel(degw):
    return pl.pallas_call(
        _dis_body,
        out_shape=jax.ShapeDtypeStruct((N_PAD, 1), jnp.float32),
    )(degw)


def _mm_body(x_ref, w_ref, o_ref):
    o_ref[0] = jnp.dot(x_ref[...], w_ref[...],
                       preferred_element_type=jnp.float32)


def _matmul_fc(x, w):
    k, dout = w.shape
    fc_chunks = dout // 128
    tm = 512
    grid = (N_PAD // tm, fc_chunks)
    return pl.pallas_call(
        _mm_body,
        grid=grid,
        in_specs=[
            pl.BlockSpec((tm, k), lambda m, f: (m, 0)),
            pl.BlockSpec((k, 128), lambda m, f: (0, f)),
        ],
        out_specs=pl.BlockSpec((1, tm, 128), lambda m, f: (f, m, 0)),
        out_shape=jax.ShapeDtypeStruct((fc_chunks, N_PAD, 128), jnp.float32),
    )(x, w)


def _epi_body(relu, agg_ref, b_ref, o_ref):
    val = agg_ref[0] + b_ref[...]
    if relu:
        val = jnp.maximum(val, 0.0)
    o_ref[...] = val


def _epilogue(aggp, bias2d, relu):
    fc_chunks = aggp.shape[0]
    dout = fc_chunks * 128
    tm = 512
    grid = (N_PAD // tm, fc_chunks)
    return pl.pallas_call(
        functools.partial(_epi_body, relu),
        grid=grid,
        in_specs=[
            pl.BlockSpec((1, tm, 128), lambda m, f: (f, m, 0)),
            pl.BlockSpec((1, 128), lambda m, f: (0, f)),
        ],
        out_specs=pl.BlockSpec((tm, 128), lambda m, f: (m, f)),
        out_shape=jax.ShapeDtypeStruct((N_PAD, dout), jnp.float32),
    )(aggp, bias2d)


def kernel(x, edge_index, edge_weights, W1, b1, W2, b2, W3, b3):
    loop = jnp.arange(N_PAD, dtype=jnp.int32)
    tail = jnp.zeros((E_PAD - E - N_PAD,), jnp.int32)
    row = jnp.concatenate([edge_index[0], loop, tail])
    col = jnp.concatenate([edge_index[1], loop, tail])
    ew = jnp.concatenate([edge_weights, jnp.ones((N_PAD,), jnp.float32),
                          tail.astype(jnp.float32)])
    row = row.reshape(TBLK, BLK)
    col = col.reshape(TBLK, BLK)
    ew = ew.reshape(TBLK, BLK)
    xp = jnp.pad(x, ((0, N_PAD - N), (0, 0)))

    degw = _prep_deg(row, col, ew)
    dis = _dis_kernel(degw)
    norm = _prep_norm(row, col, ew, dis)

    h = xp
    for (w, b, agg_k, relu) in ((W1, b1, _agg4, True),
                                (W2, b2, _agg4, True),
                                (W3, b3, _agg2, False)):
        hw = _matmul_fc(h, w)
        aggp = agg_k(hw, row, col, norm)
        h = _epilogue(aggp, b.reshape(1, -1), relu)

    return h[:N]
